# baseline (speedup 1.0000x reference)
import functools

import jax
import jax.numpy as jnp
import numpy as np
from jax import lax
from jax.experimental import pallas as pl
from jax.experimental.pallas import tpu as pltpu
from jax.experimental.pallas import tpu_sc as plsc

EPS_ = 1e-07
H_ = 2048
N_ = 16384
Q_ = 0.841

AQ_ = np.float32(Q_) * np.float32(N_ - 1)
J_UPP = int(np.floor(AQ_))
FRAC_ = float(AQ_ - np.float32(J_UPP))
RANKS = (N_ // 2 - 1, N_ // 2, J_UPP, J_UPP + 1)

NTILES = 32
CPT = H_ // NTILES
T_ = 256
NCHUNK = N_ // T_
L = 16

SIGN_ = np.uint32(0x80000000)
LOW31_ = np.uint32(0x7FFFFFFF)
TOPMASK = (None, np.uint32(0xFF000000), np.uint32(0xFFFF0000),
           np.uint32(0xFFFFFF00))


def _ukey(v):
    b = lax.bitcast_convert_type(v, jnp.uint32)
    m = b >> 31
    return b ^ (m * LOW31_) ^ SIGN_


def _ukey_to_f32(u):
    k = u ^ SIGN_
    m = k >> 31
    return lax.bitcast_convert_type(k ^ (m * LOW31_), jnp.float32)


def _sc_body(x_hbm, med_hbm, upp_hbm, sc_hbm, om_hbm, os_hbm,
             buf_v, hist_v, pref_v, rank_v, med_v, upp_v, scal_v,
             outm_v, outs_v):
    cid = lax.axis_index("c")
    sid = lax.axis_index("s")
    wid = sid * 2 + cid
    c0 = wid * CPT

    iota = lax.iota(jnp.int32, L)
    zeros_i = jnp.zeros((L,), jnp.int32)
    zeros_u = jnp.zeros((L,), jnp.uint32)
    ones_i = jnp.ones((L,), jnp.int32)

    pltpu.sync_copy(med_hbm.at[pl.ds(c0, CPT)], med_v)
    pltpu.sync_copy(upp_hbm.at[pl.ds(c0, CPT)], upp_v)
    pltpu.sync_copy(sc_hbm, scal_v)

    def zbody(i, _):
        hist_v[pl.ds(i * L, L)] = zeros_i
        return 0

    lax.fori_loop(0, (CPT * 4 * 256) // L, zbody, 0, unroll=8)

    for r in range(4):
        for g in range(CPT // L):
            sl = pl.ds(r * CPT + g * L, L)
            rank_v[sl] = jnp.full((L,), RANKS[r], jnp.int32)
            pref_v[sl] = zeros_u

    for p in range(4):
        shift = 24 - 8 * p
        if p > 0:
            lax.fori_loop(0, (CPT * 4 * 256) // L, zbody, 0, unroll=8)

        def chunk_body(ck, _, p=p, shift=shift):
            row0 = ck * T_
            pltpu.sync_copy(x_hbm.at[pl.ds(row0, T_), pl.ds(c0, CPT)],
                            buf_v)
            for g in range(CPT // L):
                chbase = (iota + g * L) * 1024
                if p > 0:
                    prefs = [pref_v[pl.ds(r * CPT + g * L, L)]
                             for r in range(4)]
                    bases = [chbase + r * 256 for r in range(4)]

                def row_body(i, _, g=g, chbase=chbase):
                    v = buf_v[i, pl.ds(g * L, L)]
                    u = _ukey(v)
                    dig = ((u >> shift) & np.uint32(0xFF)).astype(jnp.int32)
                    if p == 0:
                        plsc.addupdate_scatter(hist_v, [chbase + dig],
                                               ones_i)
                    else:
                        hi = u & TOPMASK[p]
                        for r in range(4):
                            plsc.addupdate_scatter(
                                hist_v, [bases[r] + dig], ones_i,
                                mask=hi == prefs[r])
                    return 0

                lax.fori_loop(0, T_, row_body, 0)
            return 0

        lax.fori_loop(0, NCHUNK, chunk_body, 0)

        for g in range(CPT // L):
            chbase = (iota + g * L) * 1024
            for r in range(4):
                rsl = 0 if p == 0 else r
                sl = pl.ds(r * CPT + g * L, L)
                rank = rank_v[sl]
                base = chbase + rsl * 256

                def wbody(d, carry, base=base, rank=rank):
                    cdf, dcnt, skip = carry
                    h = plsc.load_gather(hist_v, [base + d])
                    cdf2 = cdf + h
                    le = cdf2 <= rank
                    dcnt2 = dcnt + le.astype(jnp.int32)
                    skip2 = jnp.where(le, cdf2, skip)
                    return cdf2, dcnt2, skip2

                _, dcnt, skip = lax.fori_loop(
                    0, 256, wbody, (zeros_i, zeros_i, zeros_i))
                pref_v[sl] = pref_v[sl] | (dcnt.astype(jnp.uint32) << shift)
                rank_v[sl] = rank - skip

    bpow = scal_v[pl.ds(0, L)]
    dive = scal_v[pl.ds(L, L)]
    trig = scal_v[pl.ds(2 * L, L)]

    for g in range(CPT // L):
        vals = []
        for r in range(4):
            vals.append(_ukey_to_f32(pref_v[pl.ds(r * CPT + g * L, L)]))
        new_med = 0.5 * (vals[0] + vals[1])
        new_upp = vals[2] * (1.0 - FRAC_) + vals[3] * FRAC_
        med = med_v[pl.ds(g * L, L)]
        upp = upp_v[pl.ds(g * L, L)]
        med_u = bpow * med + (1.0 - bpow) * new_med
        upp_u = bpow * upp + (1.0 - bpow) * new_upp
        med_f = trig * med + (1.0 - trig) * med_u
        upp_f = trig * upp + (1.0 - trig) * upp_u
        adj_med = med_f / dive
        adj_upp = upp_f / dive
        outm_v[pl.ds(g * L, L)] = adj_med
        outs_v[pl.ds(g * L, L)] = adj_upp - adj_med + EPS_

    pltpu.sync_copy(outm_v, om_hbm.at[pl.ds(c0, CPT)])
    pltpu.sync_copy(outs_v, os_hbm.at[pl.ds(c0, CPT)])


@jax.jit
def _run(xr, med, upp, scalars):
    mesh = plsc.VectorSubcoreMesh(core_axis_name="c", subcore_axis_name="s")
    f = pl.kernel(
        _sc_body,
        mesh=mesh,
        compiler_params=pltpu.CompilerParams(use_tc_tiling_on_sc=False,
                                             needs_layout_passes=False),
        out_type=[
            jax.ShapeDtypeStruct((H_,), jnp.float32),
            jax.ShapeDtypeStruct((H_,), jnp.float32),
        ],
        scratch_types=[
            pltpu.VMEM((T_, CPT), jnp.float32),
            pltpu.VMEM((CPT * 4 * 256,), jnp.int32),
            pltpu.VMEM((4 * CPT,), jnp.uint32),
            pltpu.VMEM((4 * CPT,), jnp.int32),
            pltpu.VMEM((CPT,), jnp.float32),
            pltpu.VMEM((CPT,), jnp.float32),
            pltpu.VMEM((3 * L,), jnp.float32),
            pltpu.VMEM((CPT,), jnp.float32),
            pltpu.VMEM((CPT,), jnp.float32),
        ],
    )
    return f(xr, med, upp, scalars)


def kernel(x, med, upp, steps, beta):
    xr = x[:4].reshape(N_, H_).astype(jnp.float32)
    delta = 1.0
    bpow = beta ** delta
    trig = (steps > 1.0).astype(jnp.float32)
    steps_f = jnp.where(steps > 1.0, steps, steps + delta)
    dive = 1.0 - beta ** steps_f + EPS_
    scalars = jnp.concatenate([
        jnp.broadcast_to(bpow, (L,)),
        jnp.broadcast_to(dive, (L,)),
        jnp.broadcast_to(trig, (L,)),
    ]).astype(jnp.float32)
    out = _run(xr, med, upp, scalars)
    return (out[0], out[1])


# speedup vs baseline: 1.1183x; 1.1183x over previous
import jax
import jax.numpy as jnp
import numpy as np
from jax import lax
from jax.experimental import pallas as pl
from jax.experimental.pallas import tpu as pltpu
from jax.experimental.pallas import tpu_sc as plsc

EPS_ = 1e-07
H_ = 2048
N_ = 16384
Q_ = 0.841

AQ_ = np.float32(Q_) * np.float32(N_ - 1)
J_UPP = int(np.floor(AQ_))
FRAC_ = float(AQ_ - np.float32(J_UPP))
RANKS = (N_ // 2 - 1, N_ // 2, J_UPP, J_UPP + 1)
NR = 4

NTILES = 32
CPT = H_ // NTILES
T_ = 256
NCHUNK = N_ // T_
L = 16
NG = CPT // L
HWORDS = CPT * NR * 256

SIGN_ = np.uint32(0x80000000)
LOW31_ = np.uint32(0x7FFFFFFF)
TOPMASK = (None, np.uint32(0xFF000000), np.uint32(0xFFFF0000),
           np.uint32(0xFFFFFF00))


def _ukey(v):
    b = lax.bitcast_convert_type(v, jnp.uint32)
    m = b >> 31
    return b ^ (m * LOW31_) ^ SIGN_


def _ukey_to_f32(u):
    k = u ^ SIGN_
    m = k >> 31
    return lax.bitcast_convert_type(k ^ (m * LOW31_), jnp.float32)


def _sc_body(x_hbm, med_hbm, upp_hbm, sc_hbm, om_hbm, os_hbm,
             buf0_v, buf1_v, hist_v, pref_v, rank_v, med_v, upp_v,
             scal_v, outm_v, outs_v, sem0, sem1):
    cid = lax.axis_index("c")
    sid = lax.axis_index("s")
    wid = sid * 2 + cid
    c0 = wid * CPT

    bufs = (buf0_v, buf1_v)
    sems = (sem0, sem1)

    iota = lax.iota(jnp.int32, L)
    zeros_i = jnp.zeros((L,), jnp.int32)
    zeros_u = jnp.zeros((L,), jnp.uint32)
    ones_i = jnp.ones((L,), jnp.int32)

    def dma(ck, b):
        return pltpu.make_async_copy(
            x_hbm.at[pl.ds(ck * T_, T_), pl.ds(c0, CPT)], bufs[b], sems[b])

    dma(0, 0).start()

    pltpu.sync_copy(med_hbm.at[pl.ds(c0, CPT)], med_v)
    pltpu.sync_copy(upp_hbm.at[pl.ds(c0, CPT)], upp_v)
    pltpu.sync_copy(sc_hbm, scal_v)

    def zbody(i, _):
        hist_v[pl.ds(i * L, L)] = zeros_i
        return 0

    lax.fori_loop(0, HWORDS // L, zbody, 0, unroll=8)

    for r in range(NR):
        for g in range(NG):
            sl = pl.ds(r * CPT + g * L, L)
            rank_v[sl] = jnp.full((L,), RANKS[r], jnp.int32)
            pref_v[sl] = zeros_u

    for p in range(4):
        shift = 24 - 8 * p
        if p > 0:
            lax.fori_loop(0, HWORDS // L, zbody, 0, unroll=8)

        def process(buf, p=p, shift=shift):
            for g in range(NG):
                chbase = (iota + g * L) * (NR * 256)
                if p > 0:
                    prefs = [pref_v[pl.ds(r * CPT + g * L, L)]
                             for r in range(NR)]
                    bases = [chbase + r * 256 for r in range(NR)]

                def row_body(i, _, g=g, chbase=chbase):
                    v = buf[i, pl.ds(g * L, L)]
                    u = _ukey(v)
                    dig = ((u >> shift) & np.uint32(0xFF)).astype(jnp.int32)
                    if p == 0:
                        plsc.addupdate_scatter(hist_v, [chbase + dig],
                                               ones_i)
                    else:
                        hi = u & TOPMASK[p]
                        for r in range(NR):
                            plsc.addupdate_scatter(
                                hist_v, [bases[r] + dig], ones_i,
                                mask=hi == prefs[r])
                    return 0

                lax.fori_loop(0, T_, row_body, 0, unroll=4)

        def pair_body(ip, _, process=process, last=(p == 3)):
            for b in (0, 1):
                ck = 2 * ip + b
                if b == 0:
                    dma(ck + 1, 1).start()
                else:
                    @pl.when(ip < NCHUNK // 2 - 1)
                    def _():
                        dma(ck + 1, 0).start()
                dma(ck, b).wait()
                process(bufs[b])
            return 0

        lax.fori_loop(0, NCHUNK // 2, pair_body, 0)
        if p < 3:
            dma(0, 0).start()

        for g in range(NG):
            chbase = (iota + g * L) * (NR * 256)
            for r in range(NR):
                rsl = 0 if p == 0 else r
                sl = pl.ds(r * CPT + g * L, L)
                rank = rank_v[sl]
                base = chbase + rsl * 256

                def wbody(d, carry, base=base, rank=rank):
                    cdf, dcnt, skip = carry
                    h = plsc.load_gather(hist_v, [base + d])
                    cdf2 = cdf + h
                    le = cdf2 <= rank
                    dcnt2 = dcnt + le.astype(jnp.int32)
                    skip2 = jnp.where(le, cdf2, skip)
                    return cdf2, dcnt2, skip2

                _, dcnt, skip = lax.fori_loop(
                    0, 256, wbody, (zeros_i, zeros_i, zeros_i), unroll=4)
                pref_v[sl] = pref_v[sl] | (dcnt.astype(jnp.uint32) << shift)
                rank_v[sl] = rank - skip

    bpow = scal_v[pl.ds(0, L)]
    dive = scal_v[pl.ds(L, L)]
    trig = scal_v[pl.ds(2 * L, L)]

    for g in range(NG):
        vals = []
        for r in range(NR):
            vals.append(_ukey_to_f32(pref_v[pl.ds(r * CPT + g * L, L)]))
        new_med = 0.5 * (vals[0] + vals[1])
        new_upp = vals[2] * (1.0 - FRAC_) + vals[3] * FRAC_
        med = med_v[pl.ds(g * L, L)]
        upp = upp_v[pl.ds(g * L, L)]
        med_u = bpow * med + (1.0 - bpow) * new_med
        upp_u = bpow * upp + (1.0 - bpow) * new_upp
        med_f = trig * med + (1.0 - trig) * med_u
        upp_f = trig * upp + (1.0 - trig) * upp_u
        adj_med = med_f / dive
        adj_upp = upp_f / dive
        outm_v[pl.ds(g * L, L)] = adj_med
        outs_v[pl.ds(g * L, L)] = adj_upp - adj_med + EPS_

    pltpu.sync_copy(outm_v, om_hbm.at[pl.ds(c0, CPT)])
    pltpu.sync_copy(outs_v, os_hbm.at[pl.ds(c0, CPT)])


@jax.jit
def _run(xr, med, upp, scalars):
    mesh = plsc.VectorSubcoreMesh(core_axis_name="c", subcore_axis_name="s")
    f = pl.kernel(
        _sc_body,
        mesh=mesh,
        compiler_params=pltpu.CompilerParams(use_tc_tiling_on_sc=False,
                                             needs_layout_passes=False),
        out_type=[
            jax.ShapeDtypeStruct((H_,), jnp.float32),
            jax.ShapeDtypeStruct((H_,), jnp.float32),
        ],
        scratch_types=[
            pltpu.VMEM((T_, CPT), jnp.float32),
            pltpu.VMEM((T_, CPT), jnp.float32),
            pltpu.VMEM((HWORDS,), jnp.int32),
            pltpu.VMEM((NR * CPT,), jnp.uint32),
            pltpu.VMEM((NR * CPT,), jnp.int32),
            pltpu.VMEM((CPT,), jnp.float32),
            pltpu.VMEM((CPT,), jnp.float32),
            pltpu.VMEM((3 * L,), jnp.float32),
            pltpu.VMEM((CPT,), jnp.float32),
            pltpu.VMEM((CPT,), jnp.float32),
            pltpu.SemaphoreType.DMA,
            pltpu.SemaphoreType.DMA,
        ],
    )
    return f(xr, med, upp, scalars)


def kernel(x, med, upp, steps, beta):
    xr = x[:4].reshape(N_, H_).astype(jnp.float32)
    delta = 1.0
    bpow = beta ** delta
    trig = (steps > 1.0).astype(jnp.float32)
    steps_f = jnp.where(steps > 1.0, steps, steps + delta)
    dive = 1.0 - beta ** steps_f + EPS_
    scalars = jnp.concatenate([
        jnp.broadcast_to(bpow, (L,)),
        jnp.broadcast_to(dive, (L,)),
        jnp.broadcast_to(trig, (L,)),
    ]).astype(jnp.float32)
    out = _run(xr, med, upp, scalars)
    return (out[0], out[1])


# speedup vs baseline: 2.7972x; 2.5014x over previous
import jax
import jax.numpy as jnp
import numpy as np
from jax import lax
from jax.experimental import pallas as pl
from jax.experimental.pallas import tpu as pltpu
from jax.experimental.pallas import tpu_sc as plsc

EPS_ = 1e-07
H_ = 2048
N_ = 16384
Q_ = 0.841

AQ_ = np.float32(Q_) * np.float32(N_ - 1)
J_UPP = int(np.floor(AQ_))
FRAC_ = float(AQ_ - np.float32(J_UPP))
RANKS = (N_ // 2 - 1, N_ // 2, J_UPP, J_UPP + 1)
NR = 4

NTILES = 32
CPT = H_ // NTILES
T_ = 256
NCHUNK = N_ // T_
L = 16
NG = CPT // L
HWORDS = CPT * NR * 256

SIGN_ = np.uint32(0x80000000)
LOW31_ = np.uint32(0x7FFFFFFF)
TOPMASK = (None, np.uint32(0xFF000000), np.uint32(0xFFFF0000),
           np.uint32(0xFFFFFF00))


def _ukey(v):
    b = lax.bitcast_convert_type(v, jnp.uint32)
    m = b >> 31
    return b ^ (m * LOW31_) ^ SIGN_


def _ukey_to_f32(u):
    k = u ^ SIGN_
    m = k >> 31
    return lax.bitcast_convert_type(k ^ (m * LOW31_), jnp.float32)


def _sc_body(x_hbm, med_hbm, upp_hbm, sc_hbm, om_hbm, os_hbm,
             buf0_v, buf1_v, hist_v, pref_v, rank_v, med_v, upp_v,
             scal_v, outm_v, outs_v, sem0, sem1):
    cid = lax.axis_index("c")
    sid = lax.axis_index("s")
    wid = sid * 2 + cid
    c0 = wid * CPT

    bufs = (buf0_v, buf1_v)
    sems = (sem0, sem1)

    iota = lax.iota(jnp.int32, L)
    zeros_i = jnp.zeros((L,), jnp.int32)
    zeros_u = jnp.zeros((L,), jnp.uint32)
    ones_i = jnp.ones((L,), jnp.int32)

    def dma(ck, b):
        return pltpu.make_async_copy(
            x_hbm.at[pl.ds(ck * T_, T_), pl.ds(c0, CPT)], bufs[b], sems[b])

    dma(0, 0).start()

    pltpu.sync_copy(med_hbm.at[pl.ds(c0, CPT)], med_v)
    pltpu.sync_copy(upp_hbm.at[pl.ds(c0, CPT)], upp_v)
    pltpu.sync_copy(sc_hbm, scal_v)

    def zero_hist():
        @plsc.parallel_loop(0, HWORDS, step=L, unroll=8)
        def _(i):
            hist_v[pl.ds(i, L)] = zeros_i

    zero_hist()

    for r in range(NR):
        for g in range(NG):
            sl = pl.ds(r * CPT + g * L, L)
            rank_v[sl] = jnp.full((L,), RANKS[r], jnp.int32)
            pref_v[sl] = zeros_u

    for p in range(4):
        shift = 24 - 8 * p
        if p > 0:
            zero_hist()

        def process(buf, p=p, shift=shift):
            for g in range(NG):
                chbase = (iota + g * L) * (NR * 256)
                if p > 0:
                    prefs = [pref_v[pl.ds(r * CPT + g * L, L)]
                             for r in range(NR)]
                    bases = [chbase + r * 256 for r in range(NR)]

                @plsc.parallel_loop(0, T_, unroll=4)
                def row_body(i, g=g, chbase=chbase):
                    v = buf[i, pl.ds(g * L, L)]
                    u = _ukey(v)
                    dig = ((u >> shift) & np.uint32(0xFF)).astype(jnp.int32)
                    if p == 0:
                        plsc.addupdate_scatter(hist_v, [chbase + dig],
                                               ones_i)
                    else:
                        hi = u & TOPMASK[p]
                        for r in range(NR):
                            plsc.addupdate_scatter(
                                hist_v, [bases[r] + dig], ones_i,
                                mask=hi == prefs[r])

        def pair_body(ip, _, process=process, last=(p == 3)):
            for b in (0, 1):
                ck = 2 * ip + b
                if b == 0:
                    dma(ck + 1, 1).start()
                else:
                    @pl.when(ip < NCHUNK // 2 - 1)
                    def _():
                        dma(ck + 1, 0).start()
                dma(ck, b).wait()
                process(bufs[b])
            return 0

        lax.fori_loop(0, NCHUNK // 2, pair_body, 0)
        if p < 3:
            dma(0, 0).start()

        for g in range(NG):
            chbase = (iota + g * L) * (NR * 256)
            for r in range(NR):
                rsl = 0 if p == 0 else r
                sl = pl.ds(r * CPT + g * L, L)
                rank = rank_v[sl]
                base = chbase + rsl * 256

                def wbody(d, carry, base=base, rank=rank):
                    cdf, dcnt, skip = carry
                    h = plsc.load_gather(hist_v, [base + d])
                    cdf2 = cdf + h
                    le = cdf2 <= rank
                    dcnt2 = dcnt + le.astype(jnp.int32)
                    skip2 = jnp.where(le, cdf2, skip)
                    return cdf2, dcnt2, skip2

                _, dcnt, skip = lax.fori_loop(
                    0, 256, wbody, (zeros_i, zeros_i, zeros_i), unroll=4)
                pref_v[sl] = pref_v[sl] | (dcnt.astype(jnp.uint32) << shift)
                rank_v[sl] = rank - skip

    bpow = scal_v[pl.ds(0, L)]
    dive = scal_v[pl.ds(L, L)]
    trig = scal_v[pl.ds(2 * L, L)]

    for g in range(NG):
        vals = []
        for r in range(NR):
            vals.append(_ukey_to_f32(pref_v[pl.ds(r * CPT + g * L, L)]))
        new_med = 0.5 * (vals[0] + vals[1])
        new_upp = vals[2] * (1.0 - FRAC_) + vals[3] * FRAC_
        med = med_v[pl.ds(g * L, L)]
        upp = upp_v[pl.ds(g * L, L)]
        med_u = bpow * med + (1.0 - bpow) * new_med
        upp_u = bpow * upp + (1.0 - bpow) * new_upp
        med_f = trig * med + (1.0 - trig) * med_u
        upp_f = trig * upp + (1.0 - trig) * upp_u
        adj_med = med_f / dive
        adj_upp = upp_f / dive
        outm_v[pl.ds(g * L, L)] = adj_med
        outs_v[pl.ds(g * L, L)] = adj_upp - adj_med + EPS_

    pltpu.sync_copy(outm_v, om_hbm.at[pl.ds(c0, CPT)])
    pltpu.sync_copy(outs_v, os_hbm.at[pl.ds(c0, CPT)])


@jax.jit
def _run(xr, med, upp, scalars):
    mesh = plsc.VectorSubcoreMesh(core_axis_name="c", subcore_axis_name="s")
    f = pl.kernel(
        _sc_body,
        mesh=mesh,
        compiler_params=pltpu.CompilerParams(use_tc_tiling_on_sc=False,
                                             needs_layout_passes=False),
        out_type=[
            jax.ShapeDtypeStruct((H_,), jnp.float32),
            jax.ShapeDtypeStruct((H_,), jnp.float32),
        ],
        scratch_types=[
            pltpu.VMEM((T_, CPT), jnp.float32),
            pltpu.VMEM((T_, CPT), jnp.float32),
            pltpu.VMEM((HWORDS,), jnp.int32),
            pltpu.VMEM((NR * CPT,), jnp.uint32),
            pltpu.VMEM((NR * CPT,), jnp.int32),
            pltpu.VMEM((CPT,), jnp.float32),
            pltpu.VMEM((CPT,), jnp.float32),
            pltpu.VMEM((3 * L,), jnp.float32),
            pltpu.VMEM((CPT,), jnp.float32),
            pltpu.VMEM((CPT,), jnp.float32),
            pltpu.SemaphoreType.DMA,
            pltpu.SemaphoreType.DMA,
        ],
    )
    return f(xr, med, upp, scalars)


def kernel(x, med, upp, steps, beta):
    xr = x[:4].reshape(N_, H_).astype(jnp.float32)
    delta = 1.0
    bpow = beta ** delta
    trig = (steps > 1.0).astype(jnp.float32)
    steps_f = jnp.where(steps > 1.0, steps, steps + delta)
    dive = 1.0 - beta ** steps_f + EPS_
    scalars = jnp.concatenate([
        jnp.broadcast_to(bpow, (L,)),
        jnp.broadcast_to(dive, (L,)),
        jnp.broadcast_to(trig, (L,)),
    ]).astype(jnp.float32)
    out = _run(xr, med, upp, scalars)
    return (out[0], out[1])
